# baseline (speedup 1.0000x reference)
import numpy as np
import jax
import jax.numpy as jnp
from jax import lax
from jax.experimental import pallas as pl
from jax.experimental.pallas import tpu as pltpu

_C, _H, _W = 3, 2048, 2048
_NUM_CRACKS = 100
_MAX_LENGTH = 2
_MAX_WIDTH = 2


def _scratch_mask_np(cols, rows, seed=0):
    rng = np.random.default_rng(seed)
    n = int(rng.integers(1, _NUM_CRACKS))
    x_start = rng.integers(0, rows, size=n)
    x_end = rng.integers(0, rows, size=n)
    y_start = rng.integers(0, cols, size=n)
    y_end = rng.integers(0, cols, size=n)
    length = rng.integers(1, _MAX_LENGTH, size=n)
    width = rng.integers(1, _MAX_WIDTH, size=n)
    mask = np.zeros((cols, rows), dtype=bool)
    for i in range(n):
        xs, xe = int(x_start[i]), int(x_end[i])
        ys, ye = int(y_start[i]), int(y_end[i])
        l, w = int(length[i]), int(width[i])
        dx, dy = abs(xe - xs), abs(ye - ys)
        sx = 1 if xs < xe else -1
        sy = 1 if ys < ye else -1
        err = dx - dy
        while xs != xe or ys != ye:
            mask[ys:ys + w, xs:xs + l] = True
            e2 = 2 * err
            if e2 > -dy:
                err -= dy
                xs += sx
            if e2 < dx:
                err += dx
                ys += sy
    return mask


_MASK_NP = np.where(_scratch_mask_np(_H, _W), np.float32(np.inf),
                    np.float32(-np.inf))

_ROWS = _C * _H
_NB = 24
_BR = _ROWS // _NB
_BPC = _H // _BR


def _body(x_hbm, mask_hbm, o_hbm, img_v, mask_v, sem_in, sem_out, sem_msk):
    pltpu.make_async_copy(mask_hbm, mask_v, sem_msk).start()
    for b in range(_NB):
        pltpu.make_async_copy(
            x_hbm.at[pl.ds(b * _BR, _BR), :],
            img_v.at[pl.ds(b * _BR, _BR), :],
            sem_in.at[b],
        ).start()

    def _reduce(b, m):
        pltpu.make_async_copy(
            x_hbm.at[pl.ds(b * _BR, _BR), :],
            img_v.at[pl.ds(b * _BR, _BR), :],
            sem_in.at[b],
        ).wait()
        return jnp.maximum(m, jnp.max(img_v[pl.ds(b * _BR, _BR), :]))

    val = lax.fori_loop(0, _NB, _reduce, -jnp.inf, unroll=True)

    pltpu.make_async_copy(mask_hbm, mask_v, sem_msk).wait()

    def _writeback(b, _):
        r = lax.rem(b, _BPC) * _BR
        clip = jnp.minimum(mask_v[pl.ds(r, _BR), :].astype(jnp.float32), val)
        img_v[pl.ds(b * _BR, _BR), :] = jnp.maximum(
            img_v[pl.ds(b * _BR, _BR), :], clip
        )
        pltpu.make_async_copy(
            img_v.at[pl.ds(b * _BR, _BR), :],
            o_hbm.at[pl.ds(b * _BR, _BR), :],
            sem_out.at[b],
        ).start()
        return 0

    lax.fori_loop(0, _NB, _writeback, 0, unroll=True)

    def _drain(b, _):
        pltpu.make_async_copy(
            img_v.at[pl.ds(b * _BR, _BR), :],
            o_hbm.at[pl.ds(b * _BR, _BR), :],
            sem_out.at[b],
        ).wait()
        return 0

    lax.fori_loop(0, _NB, _drain, 0, unroll=True)


_overlay = pl.pallas_call(
    _body,
    in_specs=[
        pl.BlockSpec(memory_space=pl.ANY),
        pl.BlockSpec(memory_space=pl.ANY),
    ],
    out_specs=pl.BlockSpec(memory_space=pl.ANY),
    out_shape=jax.ShapeDtypeStruct((_ROWS, _W), jnp.float32),
    scratch_shapes=[
        pltpu.VMEM((_ROWS, _W), jnp.float32),
        pltpu.VMEM((_H, _W), jnp.bfloat16),
        pltpu.SemaphoreType.DMA((_NB,)),
        pltpu.SemaphoreType.DMA((_NB,)),
        pltpu.SemaphoreType.DMA,
    ],
    compiler_params=pltpu.CompilerParams(
        vmem_limit_bytes=64 * 1024 * 1024,
    ),
)


def kernel(inp):
    out = _overlay(
        inp.reshape(_ROWS, _W), jnp.asarray(_MASK_NP, dtype=jnp.bfloat16)
    )
    return out.reshape(_C, _H, _W)


# speedup vs baseline: 1.0626x; 1.0626x over previous
import numpy as np
import jax
import jax.numpy as jnp
from jax import lax
from jax.experimental import pallas as pl
from jax.experimental.pallas import tpu as pltpu

_C, _H, _W = 3, 2048, 2048
_NUM_CRACKS = 100
_MAX_LENGTH = 2
_MAX_WIDTH = 2


def _scratch_mask_np(cols, rows, seed=0):
    rng = np.random.default_rng(seed)
    n = int(rng.integers(1, _NUM_CRACKS))
    x_start = rng.integers(0, rows, size=n)
    x_end = rng.integers(0, rows, size=n)
    y_start = rng.integers(0, cols, size=n)
    y_end = rng.integers(0, cols, size=n)
    length = rng.integers(1, _MAX_LENGTH, size=n)
    width = rng.integers(1, _MAX_WIDTH, size=n)
    mask = np.zeros((cols, rows), dtype=bool)
    for i in range(n):
        xs, xe = int(x_start[i]), int(x_end[i])
        ys, ye = int(y_start[i]), int(y_end[i])
        l, w = int(length[i]), int(width[i])
        dx, dy = abs(xe - xs), abs(ye - ys)
        sx = 1 if xs < xe else -1
        sy = 1 if ys < ye else -1
        err = dx - dy
        while xs != xe or ys != ye:
            mask[ys:ys + w, xs:xs + l] = True
            e2 = 2 * err
            if e2 > -dy:
                err -= dy
                xs += sx
            if e2 < dx:
                err += dx
                ys += sy
    return mask


_MASK_NP = _scratch_mask_np(_H, _W).astype(np.int8)

_ROWS = _C * _H
_NB = 24
_BR = _ROWS // _NB
_BPC = _H // _BR


def _img_copy(x_hbm, img_v, sem_in, b):
    return pltpu.make_async_copy(
        x_hbm.at[pl.ds(b * _BR, _BR), :],
        img_v.at[pl.ds(b * _BR, _BR), :],
        sem_in.at[b],
    )


def _msk_copy(mask_hbm, mask_v, sem_msk, s):
    return pltpu.make_async_copy(
        mask_hbm.at[pl.ds(s * _BR, _BR), :],
        mask_v.at[pl.ds(s * _BR, _BR), :],
        sem_msk.at[s],
    )


def _out_copy(img_v, o_hbm, sem_out, b):
    return pltpu.make_async_copy(
        img_v.at[pl.ds(b * _BR, _BR), :],
        o_hbm.at[pl.ds(b * _BR, _BR), :],
        sem_out.at[b],
    )


def _body(x_hbm, mask_hbm, o_hbm, img_v, mask_v, sem_in, sem_out, sem_msk):
    for b in range(_NB):
        _img_copy(x_hbm, img_v, sem_in, b).start()
    for s in range(_BPC):
        _msk_copy(mask_hbm, mask_v, sem_msk, s).start()

    val = -jnp.inf
    for b in range(_NB):
        _img_copy(x_hbm, img_v, sem_in, b).wait()
        val = jnp.maximum(val, jnp.max(img_v[pl.ds(b * _BR, _BR), :]))

    for b in range(_NB):
        s = b % _BPC
        if b < _BPC:
            _msk_copy(mask_hbm, mask_v, sem_msk, s).wait()
        mb = mask_v[pl.ds(s * _BR, _BR), :] != 0
        img_v[pl.ds(b * _BR, _BR), :] = jnp.where(
            mb, val, img_v[pl.ds(b * _BR, _BR), :]
        )
        _out_copy(img_v, o_hbm, sem_out, b).start()

    for b in range(_NB):
        _out_copy(img_v, o_hbm, sem_out, b).wait()


_overlay = pl.pallas_call(
    _body,
    in_specs=[
        pl.BlockSpec(memory_space=pl.ANY),
        pl.BlockSpec(memory_space=pl.ANY),
    ],
    out_specs=pl.BlockSpec(memory_space=pl.ANY),
    out_shape=jax.ShapeDtypeStruct((_ROWS, _W), jnp.float32),
    scratch_shapes=[
        pltpu.VMEM((_ROWS, _W), jnp.float32),
        pltpu.VMEM((_H, _W), jnp.int8),
        pltpu.SemaphoreType.DMA((_NB,)),
        pltpu.SemaphoreType.DMA((_NB,)),
        pltpu.SemaphoreType.DMA((_BPC,)),
    ],
    compiler_params=pltpu.CompilerParams(
        vmem_limit_bytes=64 * 1024 * 1024,
    ),
)


def kernel(inp):
    out = _overlay(inp.reshape(_ROWS, _W), jnp.asarray(_MASK_NP))
    return out.reshape(_C, _H, _W)


# speedup vs baseline: 1.0831x; 1.0193x over previous
import numpy as np
import jax
import jax.numpy as jnp
from jax import lax
from jax.experimental import pallas as pl
from jax.experimental.pallas import tpu as pltpu

_C, _H, _W = 3, 2048, 2048
_NUM_CRACKS = 100
_MAX_LENGTH = 2
_MAX_WIDTH = 2


def _scratch_mask_np(cols, rows, seed=0):
    rng = np.random.default_rng(seed)
    n = int(rng.integers(1, _NUM_CRACKS))
    x_start = rng.integers(0, rows, size=n)
    x_end = rng.integers(0, rows, size=n)
    y_start = rng.integers(0, cols, size=n)
    y_end = rng.integers(0, cols, size=n)
    length = rng.integers(1, _MAX_LENGTH, size=n)
    width = rng.integers(1, _MAX_WIDTH, size=n)
    mask = np.zeros((cols, rows), dtype=bool)
    for i in range(n):
        xs, xe = int(x_start[i]), int(x_end[i])
        ys, ye = int(y_start[i]), int(y_end[i])
        l, w = int(length[i]), int(width[i])
        dx, dy = abs(xe - xs), abs(ye - ys)
        sx = 1 if xs < xe else -1
        sy = 1 if ys < ye else -1
        err = dx - dy
        while xs != xe or ys != ye:
            mask[ys:ys + w, xs:xs + l] = True
            e2 = 2 * err
            if e2 > -dy:
                err -= dy
                xs += sx
            if e2 < dx:
                err += dx
                ys += sy
    return mask


_MASK_NP = _scratch_mask_np(_H, _W).astype(np.int8)

_ROWS = _C * _H
_NB = 16
_BR = _ROWS // _NB
_BPC = _H // _BR


def _img_copy(x_hbm, img_v, sem_in, b):
    return pltpu.make_async_copy(
        x_hbm.at[pl.ds(b * _BR, _BR), :],
        img_v.at[pl.ds(b * _BR, _BR), :],
        sem_in.at[b],
    )


def _msk_copy(mask_hbm, mask_v, sem_msk, s):
    return pltpu.make_async_copy(
        mask_hbm.at[pl.ds(s * _BR, _BR), :],
        mask_v.at[pl.ds(s * _BR, _BR), :],
        sem_msk.at[s],
    )


def _out_copy(img_v, o_hbm, sem_out, b):
    return pltpu.make_async_copy(
        img_v.at[pl.ds(b * _BR, _BR), :],
        o_hbm.at[pl.ds(b * _BR, _BR), :],
        sem_out.at[b],
    )


def _body(x_hbm, mask_hbm, o_hbm, img_v, mask_v, sem_in, sem_out, sem_msk):
    for b in range(_NB):
        _img_copy(x_hbm, img_v, sem_in, b).start()
    for s in range(_BPC):
        _msk_copy(mask_hbm, mask_v, sem_msk, s).start()

    val = -jnp.inf
    for b in range(_NB):
        _img_copy(x_hbm, img_v, sem_in, b).wait()
        val = jnp.maximum(val, jnp.max(img_v[pl.ds(b * _BR, _BR), :]))

    for b in range(_NB):
        s = b % _BPC
        if b < _BPC:
            _msk_copy(mask_hbm, mask_v, sem_msk, s).wait()
        mb = mask_v[pl.ds(s * _BR, _BR), :] != 0
        img_v[pl.ds(b * _BR, _BR), :] = jnp.where(
            mb, val, img_v[pl.ds(b * _BR, _BR), :]
        )
        _out_copy(img_v, o_hbm, sem_out, b).start()

    for b in range(_NB):
        _out_copy(img_v, o_hbm, sem_out, b).wait()


_overlay = pl.pallas_call(
    _body,
    in_specs=[
        pl.BlockSpec(memory_space=pl.ANY),
        pl.BlockSpec(memory_space=pl.ANY),
    ],
    out_specs=pl.BlockSpec(memory_space=pl.ANY),
    out_shape=jax.ShapeDtypeStruct((_ROWS, _W), jnp.float32),
    scratch_shapes=[
        pltpu.VMEM((_ROWS, _W), jnp.float32),
        pltpu.VMEM((_H, _W), jnp.int8),
        pltpu.SemaphoreType.DMA((_NB,)),
        pltpu.SemaphoreType.DMA((_NB,)),
        pltpu.SemaphoreType.DMA((_BPC,)),
    ],
    compiler_params=pltpu.CompilerParams(
        vmem_limit_bytes=64 * 1024 * 1024,
    ),
)


def kernel(inp):
    out = _overlay(inp.reshape(_ROWS, _W), jnp.asarray(_MASK_NP))
    return out.reshape(_C, _H, _W)
